# baseline (speedup 1.0000x reference)
import functools

import jax
import jax.numpy as jnp
from jax import lax
from jax.experimental import pallas as pl
from jax.experimental.pallas import tpu as pltpu
from jax.experimental.pallas import tpu_sc as plsc

_NC = 2
_NS = 16
_NW = _NC * _NS

_CP = pltpu.CompilerParams(
    use_tc_tiling_on_sc=True, needs_layout_passes=False, disable_bounds_checks=True
)


@functools.lru_cache(maxsize=None)
def _make_relayout(d, v, blk):
    full = (v // blk) * blk
    n_full = full // blk
    n_iter = (n_full + _NW - 1) // _NW
    tail = v - full
    mesh = plsc.VectorSubcoreMesh(core_axis_name="c", subcore_axis_name="s")

    @functools.partial(
        pl.kernel,
        mesh=mesh,
        out_type=jax.ShapeDtypeStruct((v // 2, 2 * d), jnp.float32),
        scratch_types=[
            [pltpu.VMEM((d, blk), jnp.float32)] * 2,
            [pltpu.VMEM((blk // 2, 2 * d), jnp.float32)] * 2,
            [pltpu.SemaphoreType.DMA] * 2,
            [pltpu.SemaphoreType.DMA] * 2,
        ],
        compiler_params=_CP,
    )
    def relayout(wt, wtail, out, in_v, tr_v, isems, wsems):
        wid = lax.axis_index("s") * _NC + lax.axis_index("c")
        lane = lax.iota(jnp.int32, 16)
        half = lane & 1
        mvec = lane >> 1
        rots = [(lane + j) & 15 for j in range(16)]
        dhalf = d * half

        def transpose_block(src, dst, npair):
            @plsc.parallel_loop(0, npair // 8)
            def _(i):
                pvec = i * 8 + mvec
                colv = 2 * (i * 8) + lane
                for f0 in range(0, d, 16):
                    qbase = dhalf + f0
                    for j in range(16):
                        featv = rots[j] + f0
                        vals = plsc.load_gather(src, [featv, colv])
                        plsc.store_scatter(dst, [pvec, rots[j] + qbase], vals)

        def blk_body(i, carry):
            bids = [wid * n_iter + 2 * i + b for b in range(2)]
            c0s = [pl.multiple_of(bid * blk, 128) for bid in bids]
            r0s = [pl.multiple_of(bid * (blk // 2), 8) for bid in bids]
            for b in range(2):
                @pl.when(bids[b] < n_full)
                def _(b=b):
                    pltpu.async_copy(
                        wt.at[:, pl.ds(c0s[b], blk)], in_v[b], isems[b]
                    )

            for b in range(2):
                @pl.when(bids[b] < n_full)
                def _(b=b):
                    pltpu.make_async_copy(
                        wt.at[:, pl.ds(c0s[b], blk)], in_v[b], isems[b]
                    ).wait()
                    transpose_block(in_v[b], tr_v[b], blk // 2)
                    pltpu.async_copy(
                        tr_v[b], out.at[pl.ds(r0s[b], blk // 2), :], wsems[b]
                    )

            for b in range(2):
                @pl.when(bids[b] < n_full)
                def _(b=b):
                    pltpu.make_async_copy(
                        tr_v[b], out.at[pl.ds(r0s[b], blk // 2), :], wsems[b]
                    ).wait()

            return carry

        lax.fori_loop(0, (n_iter + 1) // 2, blk_body, 0)

        if tail:
            @pl.when(wid == _NW - 1)
            def _():
                pltpu.async_copy(
                    wtail, in_v[0].at[:, pl.ds(0, 128)], isems[0]
                ).wait()
                transpose_block(in_v[0], tr_v[0], tail // 2)
                pltpu.async_copy(
                    tr_v[0].at[pl.ds(0, tail // 2), :],
                    out.at[pl.ds(full // 2, tail // 2), :],
                    wsems[0],
                ).wait()

    return relayout


@functools.lru_cache(maxsize=None)
def _make_lookup(nh, nb, d, bc):
    n_chunks = nb // bc
    per_w = n_chunks // _NW
    assert n_chunks % _NW == 0 and nh % 2 == 0 and d % 16 == 0
    mesh = plsc.VectorSubcoreMesh(core_axis_name="c", subcore_axis_name="s")
    ngrp = bc // 16

    assert nh % 4 == 0
    @functools.partial(
        pl.kernel,
        mesh=mesh,
        out_type=jax.ShapeDtypeStruct((nh, d, nb), jnp.float32),
        scratch_types=[
            pltpu.VMEM((nh, bc), jnp.int32),
            pltpu.VMEM((nh * bc,), jnp.int32),
            pltpu.VMEM((nh * bc,), jnp.int32),
            [pltpu.VMEM((bc, 2 * d), jnp.float32)] * 4,
            [pltpu.VMEM((d, bc), jnp.float32)] * 4,
            pltpu.SemaphoreType.DMA,
            [pltpu.SemaphoreType.DMA] * 4,
            [pltpu.SemaphoreType.DMA] * 4,
        ],
        compiler_params=_CP,
    )
    def lookup(
        tab, idxt, out, idx_v, gid_v, par_v, rows_v, outt_v, isem, gsems, wsems
    ):
        wid = lax.axis_index("s") * _NC + lax.axis_index("c")
        lane = lax.iota(jnp.int32, 16)
        rots = [(lane + j) & 15 for j in range(16)]

        def gather_h(h, b, b0):
            o = pl.multiple_of(h * bc, 8)
            return pltpu.async_copy(
                tab.at[gid_v.at[pl.ds(o, bc)]], rows_v[b], gsems[b]
            )

        def emit_h(h, rb, ob, b0):
            @plsc.parallel_loop(0, ngrp)
            def _(g):
                n16 = g * 16 + lane
                parv = plsc.load_gather(par_v, [h * bc + n16])
                for c0 in range(0, d, 16):
                    parc = parv + c0
                    for j in range(16):
                        vals = plsc.load_gather(rows_v[rb], [n16, parc + rots[j]])
                        plsc.store_scatter(outt_v[ob], [rots[j] + c0, n16], vals)

            return pltpu.async_copy(
                outt_v[ob], out.at[h, :, pl.ds(b0, bc)], wsems[ob]
            )

        def chunk_body(k, carry):
            b0 = pl.multiple_of((wid * per_w + k) * bc, 128)
            pltpu.async_copy(idxt.at[:, pl.ds(b0, bc)], idx_v, isem).wait()
            for hh in range(nh):
                for g in range(ngrp):
                    iv = idx_v[hh, pl.ds(g * 16, 16)]
                    gid_v[pl.ds(hh * bc + g * 16, 16)] = iv >> 1
                    par_v[pl.ds(hh * bc + g * 16, 16)] = (iv & 1) * d

            def quad_body(qi, carry2):
                h0 = 4 * qi
                gs = [gather_h(h0 + j, j, b0) for j in range(4)]
                ws = []
                for j in range(4):
                    gs[j].wait()
                    ws.append(emit_h(h0 + j, j, j, b0))
                for w in ws:
                    w.wait()
                return carry2

            lax.fori_loop(0, nh // 4, quad_body, 0)
            return carry

        lax.fori_loop(0, per_w, chunk_body, 0)

    return lookup


def kernel(input_, weight):
    b, h = input_.shape
    v, d = weight.shape
    blk = 384
    tail = v % blk
    wtail = jnp.pad(weight[v - tail :, :], ((0, 128 - tail), (0, 0))).T
    tab = _make_relayout(d, v, blk)(weight.T, wtail)
    outt = _make_lookup(h, b, d, 128)(tab, input_.T)
    return jnp.transpose(outt, (2, 0, 1))


# speedup vs baseline: 1.0317x; 1.0317x over previous
import functools

import jax
import jax.numpy as jnp
from jax import lax
from jax.experimental import pallas as pl
from jax.experimental.pallas import tpu as pltpu
from jax.experimental.pallas import tpu_sc as plsc

_NC = 2
_NS = 16
_NW = _NC * _NS

_CP = pltpu.CompilerParams(
    use_tc_tiling_on_sc=True, needs_layout_passes=False, disable_bounds_checks=True
)


@functools.lru_cache(maxsize=None)
def _make_relayout(d, v, blk):
    full = (v // blk) * blk
    n_full = full // blk
    n_iter = (n_full + _NW - 1) // _NW
    tail = v - full
    mesh = plsc.VectorSubcoreMesh(core_axis_name="c", subcore_axis_name="s")

    @functools.partial(
        pl.kernel,
        mesh=mesh,
        out_type=jax.ShapeDtypeStruct((v // 2, 2 * d), jnp.float32),
        scratch_types=[
            [pltpu.VMEM((d, blk), jnp.float32)] * 2,
            [pltpu.VMEM((blk // 2, 2 * d), jnp.float32)] * 2,
            [pltpu.SemaphoreType.DMA] * 2,
            [pltpu.SemaphoreType.DMA] * 2,
        ],
        compiler_params=_CP,
    )
    def relayout(wt, wtail, out, in_v, tr_v, isems, wsems):
        wid = lax.axis_index("s") * _NC + lax.axis_index("c")
        lane = lax.iota(jnp.int32, 16)
        half = lane & 1
        mvec = lane >> 1

        def transpose_block(src, dst, npair):
            @plsc.parallel_loop(0, npair // 8)
            def _(i):
                pvec = i * 8 + mvec
                colv = 2 * (i * 8) + lane
                for f0 in range(0, d, 16):
                    for j in range(16):
                        featv = f0 + ((lane + j) & 15)
                        vals = plsc.load_gather(src, [featv, colv])
                        plsc.store_scatter(dst, [pvec, featv + d * half], vals)

        def blk_body(i, carry):
            bids = [wid * n_iter + 2 * i + b for b in range(2)]
            c0s = [pl.multiple_of(bid * blk, 128) for bid in bids]
            r0s = [pl.multiple_of(bid * (blk // 2), 8) for bid in bids]
            for b in range(2):
                @pl.when(bids[b] < n_full)
                def _(b=b):
                    pltpu.async_copy(
                        wt.at[:, pl.ds(c0s[b], blk)], in_v[b], isems[b]
                    )

            for b in range(2):
                @pl.when(bids[b] < n_full)
                def _(b=b):
                    pltpu.make_async_copy(
                        wt.at[:, pl.ds(c0s[b], blk)], in_v[b], isems[b]
                    ).wait()
                    transpose_block(in_v[b], tr_v[b], blk // 2)
                    pltpu.async_copy(
                        tr_v[b], out.at[pl.ds(r0s[b], blk // 2), :], wsems[b]
                    )

            for b in range(2):
                @pl.when(bids[b] < n_full)
                def _(b=b):
                    pltpu.make_async_copy(
                        tr_v[b], out.at[pl.ds(r0s[b], blk // 2), :], wsems[b]
                    ).wait()

            return carry

        lax.fori_loop(0, (n_iter + 1) // 2, blk_body, 0)

        if tail:
            @pl.when(wid == _NW - 1)
            def _():
                pltpu.async_copy(
                    wtail, in_v[0].at[:, pl.ds(0, 128)], isems[0]
                ).wait()
                transpose_block(in_v[0], tr_v[0], tail // 2)
                pltpu.async_copy(
                    tr_v[0].at[pl.ds(0, tail // 2), :],
                    out.at[pl.ds(full // 2, tail // 2), :],
                    wsems[0],
                ).wait()

    return relayout


@functools.lru_cache(maxsize=None)
def _make_lookup(nh, nb, d, bc):
    n_chunks = nb // bc
    per_w = n_chunks // _NW
    assert n_chunks % _NW == 0 and nh % 2 == 0 and d % 16 == 0
    mesh = plsc.VectorSubcoreMesh(core_axis_name="c", subcore_axis_name="s")
    ngrp = bc // 16

    assert nh % 4 == 0
    @functools.partial(
        pl.kernel,
        mesh=mesh,
        out_type=jax.ShapeDtypeStruct((nh, d, nb), jnp.float32),
        scratch_types=[
            pltpu.VMEM((nh, bc), jnp.int32),
            pltpu.VMEM((nh * bc,), jnp.int32),
            pltpu.VMEM((nh * bc,), jnp.int32),
            [pltpu.VMEM((bc, 2 * d), jnp.float32)] * 2,
            [pltpu.VMEM((d, bc), jnp.float32)] * 4,
            pltpu.SemaphoreType.DMA,
            [pltpu.SemaphoreType.DMA] * 2,
            [pltpu.SemaphoreType.DMA] * 4,
        ],
        compiler_params=_CP,
    )
    def lookup(
        tab, idxt, out, idx_v, gid_v, par_v, rows_v, outt_v, isem, gsems, wsems
    ):
        wid = lax.axis_index("s") * _NC + lax.axis_index("c")
        lane = lax.iota(jnp.int32, 16)

        def gather_h(h, b, b0):
            o = pl.multiple_of(h * bc, 8)
            return pltpu.async_copy(
                tab.at[gid_v.at[pl.ds(o, bc)]], rows_v[b], gsems[b]
            )

        def emit_h(h, rb, ob, b0):
            @plsc.parallel_loop(0, ngrp)
            def _(g):
                n16 = g * 16 + lane
                parv = plsc.load_gather(par_v, [h * bc + n16])
                for c0 in range(0, d, 16):
                    for j in range(16):
                        cvec = c0 + ((lane + j) & 15)
                        vals = plsc.load_gather(rows_v[rb], [n16, parv + cvec])
                        plsc.store_scatter(outt_v[ob], [cvec, n16], vals)

            return pltpu.async_copy(
                outt_v[ob], out.at[h, :, pl.ds(b0, bc)], wsems[ob]
            )

        def chunk_body(k, carry):
            b0 = pl.multiple_of((wid * per_w + k) * bc, 128)
            pltpu.async_copy(idxt.at[:, pl.ds(b0, bc)], idx_v, isem).wait()
            for hh in range(nh):
                for g in range(ngrp):
                    iv = idx_v[hh, pl.ds(g * 16, 16)]
                    gid_v[pl.ds(hh * bc + g * 16, 16)] = iv >> 1
                    par_v[pl.ds(hh * bc + g * 16, 16)] = (iv & 1) * d

            def quad_body(qi, carry2):
                h0 = 4 * qi
                g0 = gather_h(h0, 0, b0)
                g1 = gather_h(h0 + 1, 1, b0)
                g0.wait()
                w0 = emit_h(h0, 0, 0, b0)
                g2 = gather_h(h0 + 2, 0, b0)
                g1.wait()
                w1 = emit_h(h0 + 1, 1, 1, b0)
                g3 = gather_h(h0 + 3, 1, b0)
                g2.wait()
                w2 = emit_h(h0 + 2, 0, 2, b0)
                g3.wait()
                w3 = emit_h(h0 + 3, 1, 3, b0)
                w0.wait()
                w1.wait()
                w2.wait()
                w3.wait()
                return carry2

            lax.fori_loop(0, nh // 4, quad_body, 0)
            return carry

        lax.fori_loop(0, per_w, chunk_body, 0)

    return lookup


def kernel(input_, weight):
    b, h = input_.shape
    v, d = weight.shape
    blk = 384
    tail = v % blk
    wtail = jnp.pad(weight[v - tail :, :], ((0, 128 - tail), (0, 0))).T
    tab = _make_relayout(d, v, blk)(weight.T, wtail)
    outt = _make_lookup(h, b, d, 128)(tab, input_.T)
    return jnp.transpose(outt, (2, 0, 1))
